# baseline (speedup 1.0000x reference)
import functools

import jax
import jax.numpy as jnp
from jax import lax
from jax.experimental import pallas as pl
from jax.experimental.pallas import tpu as pltpu
from jax.experimental.pallas import tpu_sc as plsc

NVARS = 512
BATCH = 4096
L = 16
NVEC = NVARS // L


def _build_permute():
    info = plsc.get_sparse_core_info()
    nc, ns = info.num_cores, info.num_subcores
    nw = nc * ns
    rows_per_w = BATCH // nw
    chunk = 32
    n_chunks = rows_per_w // chunk
    half = 16
    n_halves = rows_per_w // half
    n_obuf = 4

    mesh = plsc.VectorSubcoreMesh(core_axis_name="c", subcore_axis_name="s")

    @functools.partial(
        pl.kernel,
        mesh=mesh,
        out_type=jax.ShapeDtypeStruct((BATCH, NVARS), jnp.float32),
        compiler_params=pltpu.CompilerParams(needs_layout_passes=False),
        scratch_types=[
            pltpu.VMEM((NVARS,), jnp.int32),
            pltpu.VMEM((chunk, NVARS), jnp.float32),
            pltpu.VMEM((chunk, NVARS), jnp.float32),
            pltpu.VMEM((chunk, NVARS), jnp.float32),
            pltpu.VMEM((chunk, NVARS), jnp.float32),
            pltpu.VMEM((half, NVARS), jnp.float32),
            pltpu.VMEM((half, NVARS), jnp.float32),
            pltpu.VMEM((half, NVARS), jnp.float32),
            pltpu.VMEM((half, NVARS), jnp.float32),
            pltpu.SemaphoreType.DMA,
            pltpu.SemaphoreType.DMA,
            pltpu.SemaphoreType.DMA,
        ],
    )
    def permute(x_hbm, perm_hbm, out_hbm, idx_v, in0, in1, in2, in3,
                ob0, ob1, ob2, ob3, idx_sem, in_sem, out_sem):
        wid = lax.axis_index("s") * nc + lax.axis_index("c")
        base = wid * rows_per_w

        in_bufs = [in0, in1, in2, in3]
        out_bufs = [ob0, ob1, ob2, ob3]

        idx_h = pltpu.async_copy(perm_hbm, idx_v, idx_sem)
        in_h = [
            pltpu.async_copy(
                x_hbm.at[pl.ds(base + c * chunk, chunk)], in_bufs[c], in_sem)
            for c in range(n_chunks)
        ]

        idx_h.wait()
        jgroup = 8

        def compute(h, slot):
            in_b = in_bufs[h // 2]
            out_b = out_bufs[slot]
            r0 = (h % 2) * half

            for g in range(NVEC // jgroup):
                colsg = [idx_v[pl.ds((g * jgroup + jj) * L, L)]
                         for jj in range(jgroup)]

                @plsc.parallel_loop(0, half, 1, unroll=4)
                def _row(r, _colsg=colsg, _g=g, _r0=r0):
                    rsplat = jnp.full((L,), r + _r0, jnp.int32)
                    for jj in range(jgroup):
                        gv = plsc.load_gather(in_b, [rsplat, _colsg[jj]])
                        out_b[r, pl.ds((_g * jgroup + jj) * L, L)] = gv

        out_h = [None] * n_obuf
        for h in range(n_halves):
            slot = h % n_obuf
            if h % 2 == 0:
                in_h[h // 2].wait()
            if out_h[slot] is not None:
                out_h[slot].wait()
            compute(h, slot)
            out_h[slot] = pltpu.async_copy(
                out_bufs[slot],
                out_hbm.at[pl.ds(base + h * half, half)],
                out_sem)
        for oh in out_h:
            if oh is not None:
                oh.wait()

    return permute


_permute = _build_permute()


def kernel(x, context, perm):
    y = _permute(x, perm.astype(jnp.int32))
    return y, 0


# speedup vs baseline: 1.2845x; 1.2845x over previous
import functools

import jax
import jax.numpy as jnp
from jax import lax
from jax.experimental import pallas as pl
from jax.experimental.pallas import tpu as pltpu
from jax.experimental.pallas import tpu_sc as plsc

NVARS = 512
BATCH = 4096
L = 16
NVEC = NVARS // L


def _build_permute():
    info = plsc.get_sparse_core_info()
    nc, ns = info.num_cores, info.num_subcores
    nw = nc * ns
    rows_per_w = BATCH // nw
    chunk = 64
    n_chunks = rows_per_w // chunk

    mesh = plsc.VectorSubcoreMesh(core_axis_name="c", subcore_axis_name="s")

    @functools.partial(
        pl.kernel,
        mesh=mesh,
        out_type=jax.ShapeDtypeStruct((BATCH, NVARS), jnp.float32),
        compiler_params=pltpu.CompilerParams(needs_layout_passes=False),
        scratch_types=[
            pltpu.VMEM((NVARS,), jnp.int32),
            pltpu.VMEM((chunk, NVARS), jnp.float32),
            pltpu.VMEM((chunk, NVARS), jnp.float32),
            pltpu.VMEM((chunk, NVARS), jnp.float32),
            pltpu.SemaphoreType.DMA,
            pltpu.SemaphoreType.DMA,
            pltpu.SemaphoreType.DMA,
        ],
    )
    def permute(x_hbm, perm_hbm, out_hbm, idx_v, in0, in1, ob,
                idx_sem, in_sem, out_sem):
        wid = lax.axis_index("s") * nc + lax.axis_index("c")
        base = wid * rows_per_w

        idx_h = pltpu.async_copy(perm_hbm, idx_v, idx_sem)
        in_h = [
            pltpu.async_copy(x_hbm.at[pl.ds(base, chunk)], in0, in_sem),
            pltpu.async_copy(x_hbm.at[pl.ds(base + chunk, chunk)], in1,
                             in_sem),
        ]

        idx_h.wait()
        jgroup = 8

        def compute(in_b, out_b):
            for g in range(NVEC // jgroup):
                colsg = [idx_v[pl.ds((g * jgroup + jj) * L, L)]
                         for jj in range(jgroup)]

                @plsc.parallel_loop(0, chunk, 1, unroll=4)
                def _row(r, _colsg=colsg, _g=g):
                    rsplat = jnp.full((L,), r, jnp.int32)
                    for jj in range(jgroup):
                        gv = plsc.load_gather(in_b, [rsplat, _colsg[jj]])
                        out_b[r, pl.ds((_g * jgroup + jj) * L, L)] = gv

        in_h[0].wait()
        compute(in0, ob)
        oh0 = pltpu.async_copy(ob, out_hbm.at[pl.ds(base, chunk)], out_sem)

        in_h[1].wait()
        compute(in1, in0)
        oh1 = pltpu.async_copy(in0, out_hbm.at[pl.ds(base + chunk, chunk)],
                               out_sem)
        oh0.wait()
        oh1.wait()

    return permute


_permute = _build_permute()


def kernel(x, context, perm):
    y = _permute(x, perm.astype(jnp.int32))
    return y, 0
